# Initial kernel scaffold; baseline (speedup 1.0000x reference)
import jax
import jax.numpy as jnp
from jax.experimental import pallas as pl


def kernel(x_paper, x_author, params, edge_index_writes, edge_index_cites):
    raise NotImplementedError("write your pallas kernel here")


# speedup vs baseline: 16.5188x; 16.5188x over previous
import functools
import jax
import jax.numpy as jnp
from jax import lax
from jax.experimental import pallas as pl
from jax.experimental.pallas import tpu as pltpu

N = 50000
E = 300000
HID = 128
NH = 8
DH = 16
OUTD = 349
NBLK = 1000
EBLK = 1000
PACK = 144
NPAD = 50176


def _dense_a_body(xp, xa, wip, bip, wia, bia, wq, bq, wkw, bkw, wvw, bvw,
                  wkc, bkc, wvc, bvc, hp_o, q_o, kw_o, vw_o, kc_o, vc_o):
    f32 = jnp.float32
    hp = jnp.maximum(jnp.dot(xp[...], wip[...], preferred_element_type=f32) + bip[...], 0.0)
    ha = jnp.maximum(jnp.dot(xa[...], wia[...], preferred_element_type=f32) + bia[...], 0.0)
    hp_o[...] = hp
    q_o[...] = jnp.dot(hp, wq[...], preferred_element_type=f32) + bq[...]
    kc_o[...] = jnp.dot(hp, wkc[...], preferred_element_type=f32) + bkc[...]
    vc_o[...] = jnp.dot(hp, wvc[...], preferred_element_type=f32) + bvc[...]
    kw_o[...] = jnp.dot(ha, wkw[...], preferred_element_type=f32) + bkw[...]
    vw_o[...] = jnp.dot(ha, wvw[...], preferred_element_type=f32) + bvw[...]


def _edge_b_body(ke, qe, ve, smat, out):
    f32 = jnp.float32
    al = jnp.dot(ke[...] * qe[...], smat[...], preferred_element_type=f32)
    w = jnp.exp(al)
    wexp = jnp.dot(w, smat[...].T, preferred_element_type=f32)
    wv = ve[...] * wexp
    out[...] = jnp.concatenate([wv, w, jnp.zeros_like(w)], axis=1)


def _final_c_body(accw, accc, hp, smat, wa, ba, wout, bout, beta, out):
    f32 = jnp.float32

    def seg(acc):
        wv = acc[:, :HID]
        s = acc[:, HID:HID + NH]
        sexp = jnp.dot(s, smat[...].T, preferred_element_type=f32)
        return wv / (sexp + 1e-16)

    agg = seg(accw[...]) + seg(accc[...])
    g = 0.5 * agg * (1.0 + lax.erf(agg * 0.7071067811865476))
    o = jnp.dot(g, wa[...], preferred_element_type=f32) + ba[...]
    b = beta[0, 0]
    h2 = b * o + (1.0 - b) * hp[...]
    out[...] = jnp.dot(h2, wout[...], preferred_element_type=f32) + bout[...]


def _rowspec(blk, width):
    return pl.BlockSpec((blk, width), lambda i: (i, 0))


def _fullspec(shape):
    return pl.BlockSpec(shape, lambda i: tuple(0 for _ in shape))


def _dense_a(xp, xa, ws):
    n = xp.shape[0]
    grid = (n // NBLK,)
    outs = [jax.ShapeDtypeStruct((n, HID), jnp.float32)] * 6
    specs = [_rowspec(NBLK, HID)] * 2
    for w in ws:
        specs.append(_fullspec(w.shape))
    return pl.pallas_call(
        _dense_a_body,
        grid=grid,
        in_specs=specs,
        out_specs=[_rowspec(NBLK, HID)] * 6,
        out_shape=outs,
    )(xp, xa, *ws)


def _edge_b(ke, qe, ve, smat):
    grid = (E // EBLK,)
    return pl.pallas_call(
        _edge_b_body,
        grid=grid,
        in_specs=[_rowspec(EBLK, HID)] * 3 + [_fullspec(smat.shape)],
        out_specs=_rowspec(EBLK, PACK),
        out_shape=jax.ShapeDtypeStruct((E, PACK), jnp.float32),
    )(ke, qe, ve, smat)


def _final_c(accw, accc, hp, smat, wa, ba, wout, bout, beta):
    grid = (N // NBLK,)
    return pl.pallas_call(
        _final_c_body,
        grid=grid,
        in_specs=[_rowspec(NBLK, PACK)] * 2 + [_rowspec(NBLK, HID)]
        + [_fullspec(smat.shape), _fullspec(wa.shape), _fullspec(ba.shape),
           _fullspec(wout.shape), _fullspec(bout.shape), _fullspec((1, 1))],
        out_specs=_rowspec(NBLK, OUTD),
        out_shape=jax.ShapeDtypeStruct((N, OUTD), jnp.float32),
    )(accw, accc, hp, smat, wa, ba, wout, bout, beta)


def _fold_kv(wk, bk, wrel, scale):
    wk4 = wk.reshape(HID, NH, DH)
    bk4 = bk.reshape(NH, DH)
    wf = jnp.einsum('nhd,hdf->nhf', wk4, wrel)
    bf = jnp.einsum('hd,hdf->hf', bk4, wrel)
    if scale is not None:
        wf = wf * scale[None, :, None]
        bf = bf * scale[:, None]
    return wf.reshape(HID, HID), bf.reshape(1, HID)


def _gather_rows(table, idx):
    return jnp.take(table, idx, axis=0)


def _scatter_rows(rows, dst):
    return jax.ops.segment_sum(rows, dst, num_segments=NPAD)


def kernel(x_paper, x_author, params, edge_index_writes, edge_index_cites):
    p = params
    scale_w = p['mu_writes'] * 0.25
    scale_c = p['mu_cites'] * 0.25
    wkw, bkw = _fold_kv(p['Wk_author'], p['bk_author'], p['Watt_writes'], scale_w)
    wvw, bvw = _fold_kv(p['Wv_author'], p['bv_author'], p['Wmsg_writes'], None)
    wkc, bkc = _fold_kv(p['Wk_paper'], p['bk_paper'], p['Watt_cites'], scale_c)
    wvc, bvc = _fold_kv(p['Wv_paper'], p['bv_paper'], p['Wmsg_cites'], None)
    ws = [p['W_in_paper'], p['b_in_paper'].reshape(1, HID),
          p['W_in_author'], p['b_in_author'].reshape(1, HID),
          p['Wq_paper'], p['bq_paper'].reshape(1, HID),
          wkw, bkw, wvw, bvw, wkc, bkc, wvc, bvc]
    hp, q, kw, vw, kc, vc = _dense_a(x_paper, x_author, ws)

    smat = (jnp.arange(HID)[:, None] // DH == jnp.arange(NH)[None, :]).astype(jnp.float32)

    src_w, dst_w = edge_index_writes[0], edge_index_writes[1]
    src_c, dst_c = edge_index_cites[0], edge_index_cites[1]

    wv_w = _edge_b(_gather_rows(kw, src_w), _gather_rows(q, dst_w),
                   _gather_rows(vw, src_w), smat)
    wv_c = _edge_b(_gather_rows(kc, src_c), _gather_rows(q, dst_c),
                   _gather_rows(vc, src_c), smat)

    acc_w = _scatter_rows(wv_w, dst_w)[:N]
    acc_c = _scatter_rows(wv_c, dst_c)[:N]

    beta = jax.nn.sigmoid(p['skip_paper']).reshape(1, 1)
    return _final_c(acc_w, acc_c, hp, smat, p['Wa_paper'],
                    p['ba_paper'].reshape(1, HID), p['W_out'],
                    p['b_out'].reshape(1, OUTD), beta)


# speedup vs baseline: 45.5227x; 2.7558x over previous
import functools
import jax
import jax.numpy as jnp
from jax import lax
from jax.experimental import pallas as pl
from jax.experimental.pallas import tpu as pltpu
from jax.experimental.pallas import tpu_sc as plsc

N = 50000
E = 300000
HID = 128
NH = 8
DH = 16
OUTD = 349
NBLK = 1000
EBLK = 1000
PACK = 144
NPAD = 50176


def _dense_a_body(xp, xa, wip, bip, wia, bia, wq, bq, wkw, bkw, wvw, bvw,
                  wkc, bkc, wvc, bvc, hp_o, q_o, kw_o, vw_o, kc_o, vc_o):
    f32 = jnp.float32
    hp = jnp.maximum(jnp.dot(xp[...], wip[...], preferred_element_type=f32) + bip[...], 0.0)
    ha = jnp.maximum(jnp.dot(xa[...], wia[...], preferred_element_type=f32) + bia[...], 0.0)
    hp_o[...] = hp
    q_o[...] = jnp.dot(hp, wq[...], preferred_element_type=f32) + bq[...]
    kc_o[...] = jnp.dot(hp, wkc[...], preferred_element_type=f32) + bkc[...]
    vc_o[...] = jnp.dot(hp, wvc[...], preferred_element_type=f32) + bvc[...]
    kw_o[...] = jnp.dot(ha, wkw[...], preferred_element_type=f32) + bkw[...]
    vw_o[...] = jnp.dot(ha, wvw[...], preferred_element_type=f32) + bvw[...]


def _edge_b_body(ke, qe, ve, smat, out):
    f32 = jnp.float32
    al = jnp.dot(ke[...] * qe[...], smat[...], preferred_element_type=f32)
    w = jnp.exp(al)
    wexp = jnp.dot(w, smat[...].T, preferred_element_type=f32)
    wv = ve[...] * wexp
    out[...] = jnp.concatenate([wv, w, jnp.zeros_like(w)], axis=1)


def _final_c_body(accw, accc, hp, smat, wa, ba, wout, bout, beta, out):
    f32 = jnp.float32

    def seg(acc):
        wv = acc[:, :HID]
        s = acc[:, HID:HID + NH]
        sexp = jnp.dot(s, smat[...].T, preferred_element_type=f32)
        return wv / (sexp + 1e-16)

    agg = seg(accw[...]) + seg(accc[...])
    g = 0.5 * agg * (1.0 + lax.erf(agg * 0.7071067811865476))
    o = jnp.dot(g, wa[...], preferred_element_type=f32) + ba[...]
    b = beta[0, 0]
    h2 = b * o + (1.0 - b) * hp[...]
    out[...] = jnp.dot(h2, wout[...], preferred_element_type=f32) + bout[...]


def _rowspec(blk, width):
    return pl.BlockSpec((blk, width), lambda i: (i, 0))


def _fullspec(shape):
    return pl.BlockSpec(shape, lambda i: tuple(0 for _ in shape))


def _dense_a(xp, xa, ws):
    n = xp.shape[0]
    grid = (n // NBLK,)
    outs = [jax.ShapeDtypeStruct((n, HID), jnp.float32)] * 6
    specs = [_rowspec(NBLK, HID)] * 2
    for w in ws:
        specs.append(_fullspec(w.shape))
    return pl.pallas_call(
        _dense_a_body,
        grid=grid,
        in_specs=specs,
        out_specs=[_rowspec(NBLK, HID)] * 6,
        out_shape=outs,
    )(xp, xa, *ws)


def _edge_b(ke, qe, ve, smat):
    grid = (E // EBLK,)
    return pl.pallas_call(
        _edge_b_body,
        grid=grid,
        in_specs=[_rowspec(EBLK, HID)] * 3 + [_fullspec(smat.shape)],
        out_specs=_rowspec(EBLK, PACK),
        out_shape=jax.ShapeDtypeStruct((E, PACK), jnp.float32),
    )(ke, qe, ve, smat)


def _final_c(accw, accc, hp, smat, wa, ba, wout, bout, beta):
    grid = (N // NBLK,)
    return pl.pallas_call(
        _final_c_body,
        grid=grid,
        in_specs=[_rowspec(NBLK, PACK)] * 2 + [_rowspec(NBLK, HID)]
        + [_fullspec(smat.shape), _fullspec(wa.shape), _fullspec(ba.shape),
           _fullspec(wout.shape), _fullspec(bout.shape), _fullspec((1, 1))],
        out_specs=_rowspec(NBLK, OUTD),
        out_shape=jax.ShapeDtypeStruct((N, OUTD), jnp.float32),
    )(accw, accc, hp, smat, wa, ba, wout, bout, beta)


def _fold_kv(wk, bk, wrel, scale):
    wk4 = wk.reshape(HID, NH, DH)
    bk4 = bk.reshape(NH, DH)
    wf = jnp.einsum('nhd,hdf->nhf', wk4, wrel)
    bf = jnp.einsum('hd,hdf->hf', bk4, wrel)
    if scale is not None:
        wf = wf * scale[None, :, None]
        bf = bf * scale[:, None]
    return wf.reshape(HID, HID), bf.reshape(1, HID)


_SC_G = 600
_SC_NCH = E // _SC_G
_SC_NW = 32


def _sc_gather6(kw, q, vw, kc, vc, sw, dw, sci, dc):
    mesh = plsc.VectorSubcoreMesh(core_axis_name="c", subcore_axis_name="s")
    out_t = [jax.ShapeDtypeStruct((E, HID), jnp.float32)] * 6

    @functools.partial(
        pl.kernel, mesh=mesh, out_type=out_t,
        scratch_types=[pltpu.VMEM((_SC_G,), jnp.int32),
                       pltpu.VMEM((_SC_G, HID), jnp.float32),
                       pltpu.SemaphoreType.DMA],
    )
    def body(kw_h, q_h, vw_h, kc_h, vc_h, sw_h, dw_h, sci_h, dc_h,
             o_kw, o_qw, o_vw, o_kc, o_qc, o_vc, idx_v, rows_v, sem):
        wid = lax.axis_index("s") * 2 + lax.axis_index("c")

        def do_chunk(chunk):
            base = chunk * _SC_G
            for idx_h, pairs in ((sw_h, ((kw_h, o_kw), (vw_h, o_vw))),
                                 (dw_h, ((q_h, o_qw),)),
                                 (sci_h, ((kc_h, o_kc), (vc_h, o_vc))),
                                 (dc_h, ((q_h, o_qc),))):
                pltpu.sync_copy(idx_h.at[pl.ds(base, _SC_G)], idx_v)
                for tab, out in pairs:
                    pltpu.async_copy(tab.at[idx_v], rows_v, sem).wait()
                    pltpu.sync_copy(rows_v, out.at[pl.ds(base, _SC_G)])

        def loop_body(j, c):
            chunk = wid + j * _SC_NW

            @pl.when(chunk < _SC_NCH)
            def _():
                do_chunk(chunk)

            return c

        lax.fori_loop(0, (_SC_NCH + _SC_NW - 1) // _SC_NW, loop_body, 0)

    return body(kw, q, vw, kc, vc, sw, dw, sci, dc)


def _scatter_rows(rows, dst):
    return jax.ops.segment_sum(rows, dst, num_segments=NPAD)


def kernel(x_paper, x_author, params, edge_index_writes, edge_index_cites):
    p = params
    scale_w = p['mu_writes'] * 0.25
    scale_c = p['mu_cites'] * 0.25
    wkw, bkw = _fold_kv(p['Wk_author'], p['bk_author'], p['Watt_writes'], scale_w)
    wvw, bvw = _fold_kv(p['Wv_author'], p['bv_author'], p['Wmsg_writes'], None)
    wkc, bkc = _fold_kv(p['Wk_paper'], p['bk_paper'], p['Watt_cites'], scale_c)
    wvc, bvc = _fold_kv(p['Wv_paper'], p['bv_paper'], p['Wmsg_cites'], None)
    ws = [p['W_in_paper'], p['b_in_paper'].reshape(1, HID),
          p['W_in_author'], p['b_in_author'].reshape(1, HID),
          p['Wq_paper'], p['bq_paper'].reshape(1, HID),
          wkw, bkw, wvw, bvw, wkc, bkc, wvc, bvc]
    hp, q, kw, vw, kc, vc = _dense_a(x_paper, x_author, ws)

    smat = (jnp.arange(HID)[:, None] // DH == jnp.arange(NH)[None, :]).astype(jnp.float32)

    src_w, dst_w = edge_index_writes[0], edge_index_writes[1]
    src_c, dst_c = edge_index_cites[0], edge_index_cites[1]

    kew, qew, vew, kec, qec, vec = _sc_gather6(
        kw, q, vw, kc, vc, src_w, dst_w, src_c, dst_c)
    wv_w = _edge_b(kew, qew, vew, smat)
    wv_c = _edge_b(kec, qec, vec, smat)

    acc_w = _scatter_rows(wv_w, dst_w)[:N]
    acc_c = _scatter_rows(wv_c, dst_c)[:N]

    beta = jax.nn.sigmoid(p['skip_paper']).reshape(1, 1)
    return _final_c(acc_w, acc_c, hp, smat, p['Wa_paper'],
                    p['ba_paper'].reshape(1, HID), p['W_out'],
                    p['b_out'].reshape(1, OUTD), beta)
